# Initial kernel scaffold; baseline (speedup 1.0000x reference)
import jax
import jax.numpy as jnp
from jax.experimental import pallas as pl


def kernel(x, edge_index, edge_attr, W0, eemb0, asrc0, adst0, aedge0, W1, eemb1, asrc1, adst1, aedge1, W2, eemb2, asrc2, adst2, aedge2, W3, eemb3, asrc3, adst3, aedge3):
    raise NotImplementedError("write your pallas kernel here")


# speedup vs baseline: 66.9766x; 66.9766x over previous
import functools

import jax
import jax.numpy as jnp
from jax import lax
from jax.experimental import pallas as pl
from jax.experimental.pallas import tpu as pltpu
from jax.experimental.pallas import tpu_sc as plsc

NN = 50000
EE = 800000
HALF = NN // 2
NPAD = 50048
B = 80
EPT = EE // 16
NBLK = EPT // B
RPT = NPAD // 16
LASTR = NN - 15 * RPT
RD = 400
RF = 200
GD = NN // RD
GF = HALF // RF
F32 = jnp.float32
I32 = jnp.int32


def _elu(v):
    return jnp.where(v > 0, v, jnp.exp(jnp.minimum(v, 0.0)) - 1.0)


def _norm_h(ag, t, eemb_h):
    ssum = jnp.sum(t, axis=1, keepdims=True)
    agg = ag + jnp.dot(t, eemb_h, preferred_element_type=F32)
    return _elu(agg / (ssum + 1e-16))


def _dense_finish(h, W_ref, asrc_ref, adst_ref, ht_out, ssc_out, sdc_out, bm_out):
    ht = jnp.dot(h, W_ref[...], preferred_element_type=F32)
    h0 = ht[:, :32]
    h1 = ht[:, 32:]
    av = asrc_ref[...]
    dv = adst_ref[...]
    ss0 = jnp.sum(h0 * av[0][None, :], axis=1)
    ss1 = jnp.sum(h1 * av[1][None, :], axis=1)
    sd0 = jnp.sum(h0 * dv[0][None, :], axis=1)
    sd1 = jnp.sum(h1 * dv[1][None, :], axis=1)
    ht_out[...] = jnp.stack([h0, h1])
    ssc_out[...] = jnp.stack([ss0, ss1], axis=1)
    sdc_out[...] = jnp.stack([sd0, sd1], axis=1)
    bm_out[...] = jnp.stack(
        [jnp.max(ss0), jnp.max(ss1), jnp.max(sd0), jnp.max(sd1)]).reshape(1, 1, 4)


def _dense0_body(x_ref, W_ref, asrc_ref, adst_ref, ht_out, ssc_out, sdc_out, bm_out):
    _dense_finish(x_ref[...], W_ref, asrc_ref, adst_ref,
                  ht_out, ssc_out, sdc_out, bm_out)


def _denseN_body(ag0, ag1, t0, t1, eemb_ref, W_ref, asrc_ref, adst_ref,
                 ht_out, ssc_out, sdc_out, bm_out):
    e = eemb_ref[...]
    h = jnp.concatenate([_norm_h(ag0[...], t0[...], e[0]),
                         _norm_h(ag1[...], t1[...], e[1])], axis=1)
    _dense_finish(h, W_ref, asrc_ref, adst_ref, ht_out, ssc_out, sdc_out, bm_out)


def _final_body(agA, agB, agC, agD, tA, tB, tC, tD, eemb_ref, pred_out):
    e = eemb_ref[...]
    hu0 = _norm_h(agA[...], tA[...], e[0])
    hv0 = _norm_h(agB[...], tB[...], e[0])
    hu1 = _norm_h(agC[...], tC[...], e[1])
    hv1 = _norm_h(agD[...], tD[...], e[1])
    acc = jnp.sum(hu0 * hv0, axis=1) + jnp.sum(hu1 * hv1, axis=1)
    pred_out[...] = acc.reshape(1, 1, RF)


_DENSE_OUT = [
    jax.ShapeDtypeStruct((2, NN, 32), F32),
    jax.ShapeDtypeStruct((NN, 2), F32),
    jax.ShapeDtypeStruct((NN, 2), F32),
    jax.ShapeDtypeStruct((GD, 1, 4), F32),
]
_DENSE_OUT_SPECS = [
    pl.BlockSpec((2, RD, 32), lambda i: (0, i, 0)),
    pl.BlockSpec((RD, 2), lambda i: (i, 0)),
    pl.BlockSpec((RD, 2), lambda i: (i, 0)),
    pl.BlockSpec((1, 1, 4), lambda i: (i, 0, 0)),
]
_W_SPEC = pl.BlockSpec((64, 64), lambda i: (0, 0))
_A_SPEC = pl.BlockSpec((2, 32), lambda i: (0, 0))
_E_SPEC = pl.BlockSpec((2, 8, 32), lambda i: (0, 0, 0))

_dense0_call = pl.pallas_call(
    _dense0_body,
    grid=(GD,),
    in_specs=[pl.BlockSpec((RD, 8), lambda i: (i, 0)),
              pl.BlockSpec((8, 64), lambda i: (0, 0)),
              _A_SPEC, _A_SPEC],
    out_specs=_DENSE_OUT_SPECS,
    out_shape=_DENSE_OUT,
)

_AG_SPECS = [pl.BlockSpec((RD, 32), lambda i: (i, 0)),
             pl.BlockSpec((RD, 32), lambda i: (i + NN // RD, 0))]
_T_SPECS = [pl.BlockSpec((RD, 8), lambda i: (i, 0)),
            pl.BlockSpec((RD, 8), lambda i: (i + NN // RD, 0))]

_denseN_call = pl.pallas_call(
    _denseN_body,
    grid=(GD,),
    in_specs=_AG_SPECS + _T_SPECS + [_E_SPEC, _W_SPEC, _A_SPEC, _A_SPEC],
    out_specs=_DENSE_OUT_SPECS,
    out_shape=_DENSE_OUT,
)

_FAG = [pl.BlockSpec((RF, 32), lambda i: (i, 0)),
        pl.BlockSpec((RF, 32), lambda i: (i + HALF // RF, 0)),
        pl.BlockSpec((RF, 32), lambda i: (i + NN // RF, 0)),
        pl.BlockSpec((RF, 32), lambda i: (i + (NN + HALF) // RF, 0))]
_FT = [pl.BlockSpec((RF, 8), lambda i: (i, 0)),
       pl.BlockSpec((RF, 8), lambda i: (i + HALF // RF, 0)),
       pl.BlockSpec((RF, 8), lambda i: (i + NN // RF, 0)),
       pl.BlockSpec((RF, 8), lambda i: (i + (NN + HALF) // RF, 0))]

_final_call = pl.pallas_call(
    _final_body,
    grid=(GF,),
    in_specs=_FAG + _FT + [_E_SPEC],
    out_specs=[pl.BlockSpec((1, 1, RF), lambda i: (i, 0, 0))],
    out_shape=[jax.ShapeDtypeStruct((GF, 1, RF), F32)],
)


def _sc_body(src_hbm, dst_hbm, attr_hbm, ht_hbm, ssrc_hbm, sdst_hbm,
             se_hbm, m_hbm, za_hbm, zb_hbm,
             agg_out, t_out,
             agg_s, t_s,
             src_v, dst_v, attr_v, idxs_v, idxd_v,
             gsrc_v, gdst_v, rows_v, trow_v, se_v, m_v,
             sem1, sem2, sem3):
    c = lax.axis_index("c")
    s = lax.axis_index("s")
    r0 = s * RPT
    pltpu.sync_copy(za_hbm, agg_s.at[pl.ds(r0, RPT)])
    pltpu.sync_copy(zb_hbm, t_s.at[pl.ds(r0, RPT)])
    pltpu.sync_copy(zb_hbm.at[pl.ds(0, B)], trow_v)
    pltpu.sync_copy(se_hbm.at[pl.ds(c * 16, 16)], se_v)
    pltpu.sync_copy(m_hbm.at[pl.ds(c * 16, 16)], m_v)
    plsc.subcore_barrier()
    m16 = m_v[...]
    coff = c * NN
    eb = s * EPT
    iota16 = lax.iota(I32, 16)

    @pl.loop(0, NBLK)
    def _blk(b):
        e0 = eb + b * B
        cp1 = pltpu.async_copy(src_hbm.at[pl.ds(e0, B)], src_v, sem1)
        cp2 = pltpu.async_copy(dst_hbm.at[pl.ds(e0, B)], dst_v, sem1)
        cp3 = pltpu.async_copy(attr_hbm.at[pl.ds(e0, B)], attr_v, sem1)
        cp1.wait()
        cp2.wait()
        cp3.wait()
        for g in range(B // 16):
            sl = pl.ds(g * 16, 16)
            idxs_v[sl] = src_v[sl] + coff
            idxd_v[sl] = dst_v[sl] + coff
        g1 = pltpu.async_copy(ssrc_hbm.at[idxs_v], gsrc_v, sem2)
        g2 = pltpu.async_copy(sdst_hbm.at[idxd_v], gdst_v, sem2)
        g3 = pltpu.async_copy(ht_hbm.at[idxs_v], rows_v, sem2)
        g1.wait()
        g2.wait()
        g3.wait()
        for g in range(B // 16):
            sl = pl.ds(g * 16, 16)
            a16 = attr_v[sl]
            lg = gsrc_v[sl] + gdst_v[sl] + plsc.load_gather(se_v, [a16])
            lg = jnp.where(lg > 0, lg, lg * 0.2)
            ex = jnp.exp(lg - m16)
            plsc.store_scatter(trow_v, [iota16 + g * 16, a16], ex)
            for j in range(16):
                e = g * 16 + j
                spl = ex.at[jnp.full((16,), j, I32)].get(mode='promise_in_bounds')
                rows_v[e, pl.ds(0, 16)] = rows_v[e, pl.ds(0, 16)] * spl
                rows_v[e, pl.ds(16, 16)] = rows_v[e, pl.ds(16, 16)] * spl
        s1 = pltpu.async_copy(rows_v, agg_s.at[dst_v], sem3, add=True)
        s2 = pltpu.async_copy(trow_v, t_s.at[dst_v], sem3, add=True)
        s1.wait()
        s2.wait()
        for g in range(B // 16):
            sl = pl.ds(g * 16, 16)
            plsc.store_scatter(trow_v, [iota16 + g * 16, attr_v[sl]],
                               jnp.zeros((16,), F32))

    plsc.subcore_barrier()
    ro = c * NN + r0

    @pl.when(s < 15)
    def _():
        pltpu.sync_copy(agg_s.at[pl.ds(r0, RPT)], agg_out.at[pl.ds(ro, RPT)])
        pltpu.sync_copy(t_s.at[pl.ds(r0, RPT)], t_out.at[pl.ds(ro, RPT)])

    @pl.when(s == 15)
    def _():
        pltpu.sync_copy(agg_s.at[pl.ds(r0, LASTR)], agg_out.at[pl.ds(ro, LASTR)])
        pltpu.sync_copy(t_s.at[pl.ds(r0, LASTR)], t_out.at[pl.ds(ro, LASTR)])


@functools.lru_cache(maxsize=1)
def _get_sc_call():
  return pl.kernel(
    _sc_body,
    out_type=(jax.ShapeDtypeStruct((2 * NN, 32), F32),
              jax.ShapeDtypeStruct((2 * NN, 8), F32)),
    mesh=plsc.VectorSubcoreMesh(core_axis_name="c", subcore_axis_name="s",
                                num_cores=2, num_subcores=16),
    compiler_params=pltpu.CompilerParams(needs_layout_passes=False,
                                         use_tc_tiling_on_sc=False),
    scratch_types=[
        pltpu.VMEM_SHARED((NPAD, 32), F32),
        pltpu.VMEM_SHARED((NPAD, 8), F32),
        pltpu.VMEM((B,), I32),
        pltpu.VMEM((B,), I32),
        pltpu.VMEM((B,), I32),
        pltpu.VMEM((B,), I32),
        pltpu.VMEM((B,), I32),
        pltpu.VMEM((B,), F32),
        pltpu.VMEM((B,), F32),
        pltpu.VMEM((B, 32), F32),
        pltpu.VMEM((B, 8), F32),
        pltpu.VMEM((16,), F32),
        pltpu.VMEM((16,), F32),
        pltpu.SemaphoreType.DMA,
        pltpu.SemaphoreType.DMA,
        pltpu.SemaphoreType.DMA,
    ],
  )


def _layer_consts(eemb, aedge):
    er = eemb.reshape(5, 2, 32)
    se = jnp.sum(er * aedge[None, :, :], axis=2)
    secat = jnp.zeros((2, 16), F32).at[:, :5].set(se.T).reshape(32)
    eembt = jnp.zeros((2, 8, 32), F32).at[:, :5, :].set(er.transpose(1, 0, 2))
    return se, secat, eembt


def _mcat(bm, se):
    bmv = jnp.max(bm, axis=(0, 1))
    bound = bmv[:2] + bmv[2:] + jnp.max(se, axis=0)
    m = jnp.where(bound > 0, bound, 0.2 * bound)
    return jnp.broadcast_to(m[:, None], (2, 16)).reshape(32)


def kernel(x, edge_index, edge_attr,
           W0, eemb0, asrc0, adst0, aedge0,
           W1, eemb1, asrc1, adst1, aedge1,
           W2, eemb2, asrc2, adst2, aedge2,
           W3, eemb3, asrc3, adst3, aedge3):
    srcp = edge_index[0]
    dstp = edge_index[1]
    attrp = edge_attr
    za = jnp.zeros((RPT, 32), F32)
    zb = jnp.zeros((RPT, 8), F32)

    params = [(W0, eemb0, asrc0, adst0, aedge0),
              (W1, eemb1, asrc1, adst1, aedge1),
              (W2, eemb2, asrc2, adst2, aedge2),
              (W3, eemb3, asrc3, adst3, aedge3)]
    consts = [_layer_consts(e, ae) for (_, e, _, _, ae) in params]

    xp = jnp.pad(x, ((0, 0), (0, 4)))
    W0p = jnp.pad(W0, ((0, 4), (0, 0)))
    ht, ssc, sdc, bm = _dense0_call(xp, W0p, asrc0, adst0)

    sc_call = _get_sc_call()
    agg = t = None
    for i in range(4):
        se, secat, _ = consts[i]
        agg, t = sc_call(srcp, dstp, attrp,
                          ht.reshape(2 * NN, 32),
                          ssc.T.reshape(2 * NN),
                          sdc.T.reshape(2 * NN),
                          secat, _mcat(bm, se), za, zb)
        if i < 3:
            W = params[i + 1][0]
            asrc = params[i + 1][2]
            adst = params[i + 1][3]
            ht, ssc, sdc, bm = _denseN_call(agg, agg, t, t,
                                            consts[i][2], W, asrc, adst)

    pred = _final_call(agg, agg, agg, agg, t, t, t, t, consts[3][2])[0]
    return pred.reshape(HALF)


# speedup vs baseline: 102.1990x; 1.5259x over previous
import functools

import jax
import jax.numpy as jnp
from jax import lax
from jax.experimental import pallas as pl
from jax.experimental.pallas import tpu as pltpu
from jax.experimental.pallas import tpu_sc as plsc

NN = 50000
EE = 800000
HALF = NN // 2
NPAD = 50048
B = 48
EPT = 50112
NBLK = EPT // B
PAIRS = NBLK // 2
EPAD = EPT * 16
EALLOC = EPAD + 256
RPT = NPAD // 16
LASTR = NN - 15 * RPT
RD = 400
RF = 200
GD = NN // RD
GF = HALF // RF
F32 = jnp.float32
I32 = jnp.int32


def _elu(v):
    return jnp.where(v > 0, v, jnp.exp(jnp.minimum(v, 0.0)) - 1.0)


def _norm_h(ag, t, eemb_h):
    ssum = jnp.sum(t, axis=1, keepdims=True)
    agg = ag + jnp.dot(t, eemb_h, preferred_element_type=F32)
    return _elu(agg / (ssum + 1e-16))


def _dense_finish(h, W_ref, asrc_ref, adst_ref, ht_out, ssc_out, sdc_out, bm_out):
    ht = jnp.dot(h, W_ref[...], preferred_element_type=F32)
    h0 = ht[:, :32]
    h1 = ht[:, 32:]
    av = asrc_ref[...]
    dv = adst_ref[...]
    ss0 = jnp.sum(h0 * av[0][None, :], axis=1)
    ss1 = jnp.sum(h1 * av[1][None, :], axis=1)
    sd0 = jnp.sum(h0 * dv[0][None, :], axis=1)
    sd1 = jnp.sum(h1 * dv[1][None, :], axis=1)
    ht_out[...] = jnp.stack([h0, h1])
    ssc_out[...] = jnp.stack([ss0, ss1], axis=1)
    sdc_out[...] = jnp.stack([sd0, sd1], axis=1)
    bm_out[...] = jnp.stack(
        [jnp.max(ss0), jnp.max(ss1), jnp.max(sd0), jnp.max(sd1)]).reshape(1, 1, 4)


def _dense0_body(x_ref, W_ref, asrc_ref, adst_ref, ht_out, ssc_out, sdc_out, bm_out):
    _dense_finish(x_ref[...], W_ref, asrc_ref, adst_ref,
                  ht_out, ssc_out, sdc_out, bm_out)


def _denseN_body(ag0, ag1, t0, t1, eemb_ref, W_ref, asrc_ref, adst_ref,
                 ht_out, ssc_out, sdc_out, bm_out):
    e = eemb_ref[...]
    h = jnp.concatenate([_norm_h(ag0[...], t0[...], e[0]),
                         _norm_h(ag1[...], t1[...], e[1])], axis=1)
    _dense_finish(h, W_ref, asrc_ref, adst_ref, ht_out, ssc_out, sdc_out, bm_out)


def _final_body(agA, agB, agC, agD, tA, tB, tC, tD, eemb_ref, pred_out):
    e = eemb_ref[...]
    hu0 = _norm_h(agA[...], tA[...], e[0])
    hv0 = _norm_h(agB[...], tB[...], e[0])
    hu1 = _norm_h(agC[...], tC[...], e[1])
    hv1 = _norm_h(agD[...], tD[...], e[1])
    acc = jnp.sum(hu0 * hv0, axis=1) + jnp.sum(hu1 * hv1, axis=1)
    pred_out[...] = acc.reshape(1, 1, RF)


_DENSE_OUT = [
    jax.ShapeDtypeStruct((2, NN, 32), F32),
    jax.ShapeDtypeStruct((NN, 2), F32),
    jax.ShapeDtypeStruct((NN, 2), F32),
    jax.ShapeDtypeStruct((GD, 1, 4), F32),
]
_DENSE_OUT_SPECS = [
    pl.BlockSpec((2, RD, 32), lambda i: (0, i, 0)),
    pl.BlockSpec((RD, 2), lambda i: (i, 0)),
    pl.BlockSpec((RD, 2), lambda i: (i, 0)),
    pl.BlockSpec((1, 1, 4), lambda i: (i, 0, 0)),
]
_W_SPEC = pl.BlockSpec((64, 64), lambda i: (0, 0))
_A_SPEC = pl.BlockSpec((2, 32), lambda i: (0, 0))
_E_SPEC = pl.BlockSpec((2, 8, 32), lambda i: (0, 0, 0))

_dense0_call = pl.pallas_call(
    _dense0_body,
    grid=(GD,),
    in_specs=[pl.BlockSpec((RD, 8), lambda i: (i, 0)),
              pl.BlockSpec((8, 64), lambda i: (0, 0)),
              _A_SPEC, _A_SPEC],
    out_specs=_DENSE_OUT_SPECS,
    out_shape=_DENSE_OUT,
)

_AG_SPECS = [pl.BlockSpec((RD, 32), lambda i: (i, 0)),
             pl.BlockSpec((RD, 32), lambda i: (i + NN // RD, 0))]
_T_SPECS = [pl.BlockSpec((RD, 8), lambda i: (i, 0)),
            pl.BlockSpec((RD, 8), lambda i: (i + NN // RD, 0))]

_denseN_call = pl.pallas_call(
    _denseN_body,
    grid=(GD,),
    in_specs=_AG_SPECS + _T_SPECS + [_E_SPEC, _W_SPEC, _A_SPEC, _A_SPEC],
    out_specs=_DENSE_OUT_SPECS,
    out_shape=_DENSE_OUT,
)

_FAG = [pl.BlockSpec((RF, 32), lambda i: (i, 0)),
        pl.BlockSpec((RF, 32), lambda i: (i + HALF // RF, 0)),
        pl.BlockSpec((RF, 32), lambda i: (i + NN // RF, 0)),
        pl.BlockSpec((RF, 32), lambda i: (i + (NN + HALF) // RF, 0))]
_FT = [pl.BlockSpec((RF, 8), lambda i: (i, 0)),
       pl.BlockSpec((RF, 8), lambda i: (i + HALF // RF, 0)),
       pl.BlockSpec((RF, 8), lambda i: (i + NN // RF, 0)),
       pl.BlockSpec((RF, 8), lambda i: (i + (NN + HALF) // RF, 0))]

_final_call = pl.pallas_call(
    _final_body,
    grid=(GF,),
    in_specs=_FAG + _FT + [_E_SPEC],
    out_specs=[pl.BlockSpec((1, 1, RF), lambda i: (i, 0, 0))],
    out_shape=[jax.ShapeDtypeStruct((GF, 1, RF), F32)],
)


def _sc_body(src_hbm, dst_hbm, attr_hbm, ht_hbm, ssrc_hbm, sdst_hbm,
             se_hbm, m_hbm, za_hbm, zb_hbm,
             agg_out, t_out,
             agg_s, t_s,
             src0, src1, dst0, dst1, attr0, attr1,
             idxs0, idxs1, idxd0, idxd1, idxw0, idxw1, attrw0, attrw1,
             gsrc0, gsrc1, gdst0, gdst1, rows0, rows1, trow0, trow1,
             se_v, m_v, semL0, semL1, semG0, semG1, semS0, semS1):
    semL = (semL0, semL1)
    semG = (semG0, semG1)
    semS = (semS0, semS1)
    src_v = (src0, src1)
    dst_v = (dst0, dst1)
    attr_v = (attr0, attr1)
    idxs_v = (idxs0, idxs1)
    idxd_v = (idxd0, idxd1)
    idxw_v = (idxw0, idxw1)
    attrw_v = (attrw0, attrw1)
    gsrc_v = (gsrc0, gsrc1)
    gdst_v = (gdst0, gdst1)
    rows_v = (rows0, rows1)
    trow_v = (trow0, trow1)

    c = lax.axis_index("c")
    s = lax.axis_index("s")
    r0 = s * RPT

    @pl.when(s < 15)
    def _():
        pltpu.sync_copy(za_hbm, agg_s.at[pl.ds(r0, RPT)])
        pltpu.sync_copy(zb_hbm, t_s.at[pl.ds(r0, RPT)])

    @pl.when(s == 15)
    def _():
        pltpu.sync_copy(za_hbm.at[pl.ds(0, LASTR)], agg_s.at[pl.ds(r0, LASTR)])
        pltpu.sync_copy(zb_hbm.at[pl.ds(0, LASTR)], t_s.at[pl.ds(r0, LASTR)])

    pltpu.sync_copy(se_hbm.at[pl.ds(c * 16, 16)], se_v)
    pltpu.sync_copy(m_hbm.at[pl.ds(c * 16, 16)], m_v)
    plsc.subcore_barrier()
    m16 = m_v[...]
    coff = c * NN
    eb = s * EPT
    iota16 = lax.iota(I32, 16)
    NG = B // 16

    def fire_loads(p, b):
        e0 = eb + b * B
        pltpu.async_copy(src_hbm.at[pl.ds(e0, B)], src_v[p], semL[p])
        pltpu.async_copy(dst_hbm.at[pl.ds(e0, B)], dst_v[p], semL[p])
        pltpu.async_copy(attr_hbm.at[pl.ds(e0, B)], attr_v[p], semL[p])

    def wait_loads(p):
        pltpu.make_async_copy(src_hbm.at[pl.ds(0, B)], src_v[p], semL[p]).wait()
        pltpu.make_async_copy(dst_hbm.at[pl.ds(0, B)], dst_v[p], semL[p]).wait()
        pltpu.make_async_copy(attr_hbm.at[pl.ds(0, B)], attr_v[p], semL[p]).wait()

    def idx_compute(p):
        for g in range(NG):
            sl = pl.ds(g * 16, 16)
            dv = dst_v[p][sl]
            idxs_v[p][sl] = src_v[p][sl] + coff
            idxd_v[p][sl] = dv + coff
            idxw_v[p][sl] = dv
            attrw_v[p][sl] = attr_v[p][sl]

    def fire_gathers(p):
        pltpu.async_copy(ssrc_hbm.at[idxs_v[p]], gsrc_v[p], semG[p])
        pltpu.async_copy(sdst_hbm.at[idxd_v[p]], gdst_v[p], semG[p])
        pltpu.async_copy(ht_hbm.at[idxs_v[p]], rows_v[p], semG[p])

    def wait_gathers(p):
        pltpu.make_async_copy(ssrc_hbm.at[idxs_v[p]], gsrc_v[p], semG[p]).wait()
        pltpu.make_async_copy(sdst_hbm.at[idxd_v[p]], gdst_v[p], semG[p]).wait()
        pltpu.make_async_copy(ht_hbm.at[idxs_v[p]], rows_v[p], semG[p]).wait()

    def fire_scatters(p):
        pltpu.async_copy(rows_v[p], agg_s.at[idxw_v[p]], semS[p], add=True)
        pltpu.async_copy(trow_v[p], t_s.at[idxw_v[p]], semS[p], add=True)

    def wait_scatters(p):
        pltpu.make_async_copy(rows_v[p], agg_s.at[idxw_v[p]], semS[p]).wait()
        pltpu.make_async_copy(trow_v[p], t_s.at[idxw_v[p]], semS[p]).wait()

    def compute(p, b):
        e0 = eb + b * B
        zero16 = jnp.zeros((16,), F32)
        for g in range(NG):
            rid = iota16 + g * 16
            for cls in range(8):
                plsc.store_scatter(trow_v[p], [rid, jnp.full((16,), cls, I32)],
                                   zero16)
        for g in range(NG):
            sl = pl.ds(g * 16, 16)
            a16 = attrw_v[p][sl]
            lg = gsrc_v[p][sl] + gdst_v[p][sl] + plsc.load_gather(se_v, [a16])
            lg = jnp.where(lg > 0, lg, lg * 0.2)
            ex = jnp.exp(lg - m16)
            gid = iota16 + (e0 + g * 16)
            ex = jnp.where(gid < EE, ex, 0.0)
            plsc.store_scatter(trow_v[p], [iota16 + g * 16, a16], ex)
            for j in range(16):
                e = g * 16 + j
                spl = ex.at[jnp.full((16,), j, I32)].get(mode='promise_in_bounds')
                rows_v[p][e, pl.ds(0, 16)] = rows_v[p][e, pl.ds(0, 16)] * spl
                rows_v[p][e, pl.ds(16, 16)] = rows_v[p][e, pl.ds(16, 16)] * spl

    fire_loads(0, 0)
    fire_loads(1, 1)
    wait_loads(0)
    idx_compute(0)
    fire_gathers(0)
    fire_loads(0, 2)
    wait_loads(1)
    idx_compute(1)
    fire_gathers(1)
    fire_loads(1, 3)

    @pl.loop(0, PAIRS - 1)
    def _pair(j):
        b2 = 2 * j
        for par in (0, 1):
            wait_gathers(par)
            compute(par, b2 + par)
            fire_scatters(par)
        for par in (0, 1):
            wait_loads(par)
            wait_scatters(par)
            idx_compute(par)
            fire_gathers(par)
            fire_loads(par, b2 + 4 + par)

    for par in (0, 1):
        wait_gathers(par)
        compute(par, 2 * (PAIRS - 1) + par)
        fire_scatters(par)
    for par in (0, 1):
        wait_scatters(par)
        wait_loads(par)

    plsc.subcore_barrier()
    ro = c * NN + r0

    @pl.when(s < 15)
    def _():
        pltpu.sync_copy(agg_s.at[pl.ds(r0, RPT)], agg_out.at[pl.ds(ro, RPT)])
        pltpu.sync_copy(t_s.at[pl.ds(r0, RPT)], t_out.at[pl.ds(ro, RPT)])

    @pl.when(s == 15)
    def _():
        pltpu.sync_copy(agg_s.at[pl.ds(r0, LASTR)], agg_out.at[pl.ds(ro, LASTR)])
        pltpu.sync_copy(t_s.at[pl.ds(r0, LASTR)], t_out.at[pl.ds(ro, LASTR)])


@functools.lru_cache(maxsize=1)
def _get_sc_call():
  scr = [pltpu.VMEM_SHARED((NN, 32), F32),
         pltpu.VMEM_SHARED((NN, 8), F32)]
  scr += [pltpu.VMEM((B,), I32) for _ in range(14)]
  scr += [pltpu.VMEM((B,), F32) for _ in range(4)]
  scr += [pltpu.VMEM((B, 32), F32) for _ in range(2)]
  scr += [pltpu.VMEM((B, 8), F32) for _ in range(2)]
  scr += [pltpu.VMEM((16,), F32) for _ in range(2)]
  scr += [pltpu.SemaphoreType.DMA for _ in range(6)]
  return pl.kernel(
    _sc_body,
    out_type=(jax.ShapeDtypeStruct((2 * NN, 32), F32),
              jax.ShapeDtypeStruct((2 * NN, 8), F32)),
    mesh=plsc.VectorSubcoreMesh(core_axis_name="c", subcore_axis_name="s",
                                num_cores=2, num_subcores=16),
    compiler_params=pltpu.CompilerParams(needs_layout_passes=False,
                                         use_tc_tiling_on_sc=False),
    scratch_types=scr,
  )


def _layer_consts(eemb, aedge):
    er = eemb.reshape(5, 2, 32)
    se = jnp.sum(er * aedge[None, :, :], axis=2)
    secat = jnp.zeros((2, 16), F32).at[:, :5].set(se.T).reshape(32)
    eembt = jnp.zeros((2, 8, 32), F32).at[:, :5, :].set(er.transpose(1, 0, 2))
    return se, secat, eembt


def _mcat(bm, se):
    bmv = jnp.max(bm, axis=(0, 1))
    bound = bmv[:2] + bmv[2:] + jnp.max(se, axis=0)
    m = jnp.where(bound > 0, bound, 0.2 * bound)
    return jnp.broadcast_to(m[:, None], (2, 16)).reshape(32)


def kernel(x, edge_index, edge_attr,
           W0, eemb0, asrc0, adst0, aedge0,
           W1, eemb1, asrc1, adst1, aedge1,
           W2, eemb2, asrc2, adst2, aedge2,
           W3, eemb3, asrc3, adst3, aedge3):
    pad = EALLOC - EE
    zpad = jnp.zeros((pad,), I32)
    srcp = jnp.concatenate([edge_index[0], zpad])
    dstp = jnp.concatenate([edge_index[1], zpad])
    attrp = jnp.concatenate([edge_attr, zpad])
    za = jnp.zeros((RPT, 32), F32)
    zb = jnp.zeros((RPT, 8), F32)

    params = [(W0, eemb0, asrc0, adst0, aedge0),
              (W1, eemb1, asrc1, adst1, aedge1),
              (W2, eemb2, asrc2, adst2, aedge2),
              (W3, eemb3, asrc3, adst3, aedge3)]
    consts = [_layer_consts(e, ae) for (_, e, _, _, ae) in params]

    xp = jnp.pad(x, ((0, 0), (0, 4)))
    W0p = jnp.pad(W0, ((0, 4), (0, 0)))
    ht, ssc, sdc, bm = _dense0_call(xp, W0p, asrc0, adst0)

    sc_call = _get_sc_call()
    agg = t = None
    for i in range(4):
        se, secat, _ = consts[i]
        agg, t = sc_call(srcp, dstp, attrp,
                          ht.reshape(2 * NN, 32),
                          ssc.T.reshape(2 * NN),
                          sdc.T.reshape(2 * NN),
                          secat, _mcat(bm, se), za, zb)
        if i < 3:
            W = params[i + 1][0]
            asrc = params[i + 1][2]
            adst = params[i + 1][3]
            ht, ssc, sdc, bm = _denseN_call(agg, agg, t, t,
                                            consts[i][2], W, asrc, adst)

    pred = _final_call(agg, agg, agg, agg, t, t, t, t, consts[3][2])[0]
    return pred.reshape(HALF)


# speedup vs baseline: 118.9735x; 1.1641x over previous
import functools

import jax
import jax.numpy as jnp
from jax import lax
from jax.experimental import pallas as pl
from jax.experimental.pallas import tpu as pltpu
from jax.experimental.pallas import tpu_sc as plsc

NN = 50000
EE = 800000
HALF = NN // 2
NPAD = 50048
B = 48
EPT = 50112
NBLK = EPT // B
PAIRS = NBLK // 2
EPAD = EPT * 16
EALLOC = EPAD + 256
RPT = NPAD // 16
LASTR = NN - 15 * RPT
RD = 2000
RF = 1000
GD = NN // RD
GF = HALF // RF
F32 = jnp.float32
I32 = jnp.int32


def _elu(v):
    return jnp.where(v > 0, v, jnp.exp(jnp.minimum(v, 0.0)) - 1.0)


def _norm_h(ag, t, eemb_h):
    ssum = jnp.sum(t, axis=1, keepdims=True)
    agg = ag + jnp.dot(t, eemb_h, preferred_element_type=F32)
    return _elu(agg / (ssum + 1e-16))


def _dense_finish(h, W_ref, asrc_ref, adst_ref, ht_out, ssc_out, sdc_out, bm_out):
    ht = jnp.dot(h, W_ref[...], preferred_element_type=F32)
    h0 = ht[:, :32]
    h1 = ht[:, 32:]
    av = asrc_ref[...]
    dv = adst_ref[...]
    ss0 = jnp.sum(h0 * av[0][None, :], axis=1)
    ss1 = jnp.sum(h1 * av[1][None, :], axis=1)
    sd0 = jnp.sum(h0 * dv[0][None, :], axis=1)
    sd1 = jnp.sum(h1 * dv[1][None, :], axis=1)
    ht_out[...] = jnp.stack([h0, h1])
    ssc_out[...] = jnp.stack([ss0, ss1], axis=1)
    sdc_out[...] = jnp.stack([sd0, sd1], axis=1)
    bm_out[...] = jnp.stack(
        [jnp.max(ss0), jnp.max(ss1), jnp.max(sd0), jnp.max(sd1)]).reshape(1, 1, 4)


def _dense0_body(x_ref, W_ref, asrc_ref, adst_ref, ht_out, ssc_out, sdc_out, bm_out):
    _dense_finish(x_ref[...], W_ref, asrc_ref, adst_ref,
                  ht_out, ssc_out, sdc_out, bm_out)


def _denseN_body(ag0, ag1, t0, t1, eemb_ref, W_ref, asrc_ref, adst_ref,
                 ht_out, ssc_out, sdc_out, bm_out):
    e = eemb_ref[...]
    h = jnp.concatenate([_norm_h(ag0[...], t0[...], e[0]),
                         _norm_h(ag1[...], t1[...], e[1])], axis=1)
    _dense_finish(h, W_ref, asrc_ref, adst_ref, ht_out, ssc_out, sdc_out, bm_out)


def _final_body(agA, agB, agC, agD, tA, tB, tC, tD, eemb_ref, pred_out):
    e = eemb_ref[...]
    hu0 = _norm_h(agA[...], tA[...], e[0])
    hv0 = _norm_h(agB[...], tB[...], e[0])
    hu1 = _norm_h(agC[...], tC[...], e[1])
    hv1 = _norm_h(agD[...], tD[...], e[1])
    acc = jnp.sum(hu0 * hv0, axis=1) + jnp.sum(hu1 * hv1, axis=1)
    pred_out[...] = acc.reshape(1, 1, RF)


_DENSE_OUT = [
    jax.ShapeDtypeStruct((2, NN, 32), F32),
    jax.ShapeDtypeStruct((NN, 2), F32),
    jax.ShapeDtypeStruct((NN, 2), F32),
    jax.ShapeDtypeStruct((GD, 1, 4), F32),
]
_DENSE_OUT_SPECS = [
    pl.BlockSpec((2, RD, 32), lambda i: (0, i, 0)),
    pl.BlockSpec((RD, 2), lambda i: (i, 0)),
    pl.BlockSpec((RD, 2), lambda i: (i, 0)),
    pl.BlockSpec((1, 1, 4), lambda i: (i, 0, 0)),
]
_W_SPEC = pl.BlockSpec((64, 64), lambda i: (0, 0))
_A_SPEC = pl.BlockSpec((2, 32), lambda i: (0, 0))
_E_SPEC = pl.BlockSpec((2, 8, 32), lambda i: (0, 0, 0))

_dense0_call = pl.pallas_call(
    _dense0_body,
    grid=(GD,),
    in_specs=[pl.BlockSpec((RD, 8), lambda i: (i, 0)),
              pl.BlockSpec((8, 64), lambda i: (0, 0)),
              _A_SPEC, _A_SPEC],
    out_specs=_DENSE_OUT_SPECS,
    out_shape=_DENSE_OUT,
)

_AG_SPECS = [pl.BlockSpec((RD, 32), lambda i: (i, 0)),
             pl.BlockSpec((RD, 32), lambda i: (i + NN // RD, 0))]
_T_SPECS = [pl.BlockSpec((RD, 8), lambda i: (i, 0)),
            pl.BlockSpec((RD, 8), lambda i: (i + NN // RD, 0))]

_denseN_call = pl.pallas_call(
    _denseN_body,
    grid=(GD,),
    in_specs=_AG_SPECS + _T_SPECS + [_E_SPEC, _W_SPEC, _A_SPEC, _A_SPEC],
    out_specs=_DENSE_OUT_SPECS,
    out_shape=_DENSE_OUT,
)

_FAG = [pl.BlockSpec((RF, 32), lambda i: (i, 0)),
        pl.BlockSpec((RF, 32), lambda i: (i + HALF // RF, 0)),
        pl.BlockSpec((RF, 32), lambda i: (i + NN // RF, 0)),
        pl.BlockSpec((RF, 32), lambda i: (i + (NN + HALF) // RF, 0))]
_FT = [pl.BlockSpec((RF, 8), lambda i: (i, 0)),
       pl.BlockSpec((RF, 8), lambda i: (i + HALF // RF, 0)),
       pl.BlockSpec((RF, 8), lambda i: (i + NN // RF, 0)),
       pl.BlockSpec((RF, 8), lambda i: (i + (NN + HALF) // RF, 0))]

_final_call = pl.pallas_call(
    _final_body,
    grid=(GF,),
    in_specs=_FAG + _FT + [_E_SPEC],
    out_specs=[pl.BlockSpec((1, 1, RF), lambda i: (i, 0, 0))],
    out_shape=[jax.ShapeDtypeStruct((GF, 1, RF), F32)],
)


def _sc_body(src_hbm, dst_hbm, attr_hbm, ht_hbm, ssrc_hbm, sdst_hbm,
             se_hbm, m_hbm, za_hbm, zb_hbm,
             agg_out, t_out,
             agg_s, t_s,
             src0, src1, dst0, dst1, attr0, attr1,
             idxs0, idxs1, idxd0, idxd1, idxw0, idxw1, attrw0, attrw1,
             gsrc0, gsrc1, gdst0, gdst1, rows0, rows1, trow0, trow1,
             se_v, m_v, semL0, semL1, semG0, semG1, semS0, semS1):
    semL = (semL0, semL1)
    semG = (semG0, semG1)
    semS = (semS0, semS1)
    src_v = (src0, src1)
    dst_v = (dst0, dst1)
    attr_v = (attr0, attr1)
    idxs_v = (idxs0, idxs1)
    idxd_v = (idxd0, idxd1)
    idxw_v = (idxw0, idxw1)
    attrw_v = (attrw0, attrw1)
    gsrc_v = (gsrc0, gsrc1)
    gdst_v = (gdst0, gdst1)
    rows_v = (rows0, rows1)
    trow_v = (trow0, trow1)

    c = lax.axis_index("c")
    s = lax.axis_index("s")
    r0 = s * RPT

    @pl.when(s < 15)
    def _():
        pltpu.sync_copy(za_hbm, agg_s.at[pl.ds(r0, RPT)])
        pltpu.sync_copy(zb_hbm, t_s.at[pl.ds(r0, RPT)])

    @pl.when(s == 15)
    def _():
        pltpu.sync_copy(za_hbm.at[pl.ds(0, LASTR)], agg_s.at[pl.ds(r0, LASTR)])
        pltpu.sync_copy(zb_hbm.at[pl.ds(0, LASTR)], t_s.at[pl.ds(r0, LASTR)])

    pltpu.sync_copy(se_hbm.at[pl.ds(c * 16, 16)], se_v)
    pltpu.sync_copy(m_hbm.at[pl.ds(c * 16, 16)], m_v)
    plsc.subcore_barrier()
    m16 = m_v[...]
    coff = c * NN
    eb = s * EPT
    iota16 = lax.iota(I32, 16)
    NG = B // 16

    def fire_loads(p, b):
        e0 = eb + b * B
        pltpu.async_copy(src_hbm.at[pl.ds(e0, B)], src_v[p], semL[p])
        pltpu.async_copy(dst_hbm.at[pl.ds(e0, B)], dst_v[p], semL[p])
        pltpu.async_copy(attr_hbm.at[pl.ds(e0, B)], attr_v[p], semL[p])

    def wait_loads(p):
        pltpu.make_async_copy(src_hbm.at[pl.ds(0, B)], src_v[p], semL[p]).wait()
        pltpu.make_async_copy(dst_hbm.at[pl.ds(0, B)], dst_v[p], semL[p]).wait()
        pltpu.make_async_copy(attr_hbm.at[pl.ds(0, B)], attr_v[p], semL[p]).wait()

    def idx_compute(p):
        for g in range(NG):
            sl = pl.ds(g * 16, 16)
            dv = dst_v[p][sl]
            idxs_v[p][sl] = src_v[p][sl] + coff
            idxd_v[p][sl] = dv + coff
            idxw_v[p][sl] = dv
            attrw_v[p][sl] = attr_v[p][sl]

    def fire_gathers(p):
        pltpu.async_copy(ssrc_hbm.at[idxs_v[p]], gsrc_v[p], semG[p])
        pltpu.async_copy(sdst_hbm.at[idxd_v[p]], gdst_v[p], semG[p])
        pltpu.async_copy(ht_hbm.at[idxs_v[p]], rows_v[p], semG[p])

    def wait_gathers(p):
        pltpu.make_async_copy(ssrc_hbm.at[idxs_v[p]], gsrc_v[p], semG[p]).wait()
        pltpu.make_async_copy(sdst_hbm.at[idxd_v[p]], gdst_v[p], semG[p]).wait()
        pltpu.make_async_copy(ht_hbm.at[idxs_v[p]], rows_v[p], semG[p]).wait()

    def fire_scatters(p):
        pltpu.async_copy(rows_v[p], agg_s.at[idxw_v[p]], semS[p], add=True)
        pltpu.async_copy(trow_v[p], t_s.at[idxw_v[p]], semS[p], add=True)

    def wait_scatters(p):
        pltpu.make_async_copy(rows_v[p], agg_s.at[idxw_v[p]], semS[p]).wait()
        pltpu.make_async_copy(trow_v[p], t_s.at[idxw_v[p]], semS[p]).wait()

    def compute(p, b):
        e0 = eb + b * B
        zero16 = jnp.zeros((16,), F32)
        for g in range(NG):
            rid = iota16 + g * 16
            for cls in range(8):
                plsc.store_scatter(trow_v[p], [rid, jnp.full((16,), cls, I32)],
                                   zero16)
        for g in range(NG):
            sl = pl.ds(g * 16, 16)
            a16 = attrw_v[p][sl]
            lg = gsrc_v[p][sl] + gdst_v[p][sl] + plsc.load_gather(se_v, [a16])
            lg = jnp.where(lg > 0, lg, lg * 0.2)
            ex = jnp.exp(lg - m16)
            gid = iota16 + (e0 + g * 16)
            ex = jnp.where(gid < EE, ex, 0.0)
            plsc.store_scatter(trow_v[p], [iota16 + g * 16, a16], ex)
            for j in range(16):
                e = g * 16 + j
                spl = ex.at[jnp.full((16,), j, I32)].get(mode='promise_in_bounds')
                rows_v[p][e, pl.ds(0, 16)] = rows_v[p][e, pl.ds(0, 16)] * spl
                rows_v[p][e, pl.ds(16, 16)] = rows_v[p][e, pl.ds(16, 16)] * spl

    fire_loads(0, 0)
    fire_loads(1, 1)
    wait_loads(0)
    idx_compute(0)
    fire_gathers(0)
    fire_loads(0, 2)
    wait_loads(1)
    idx_compute(1)
    fire_gathers(1)
    fire_loads(1, 3)

    @pl.loop(0, PAIRS - 1)
    def _pair(j):
        b2 = 2 * j
        for par in (0, 1):
            wait_gathers(par)
            compute(par, b2 + par)
            fire_scatters(par)
        for par in (0, 1):
            wait_loads(par)
            wait_scatters(par)
            idx_compute(par)
            fire_gathers(par)
            fire_loads(par, b2 + 4 + par)

    for par in (0, 1):
        wait_gathers(par)
        compute(par, 2 * (PAIRS - 1) + par)
        fire_scatters(par)
    for par in (0, 1):
        wait_scatters(par)
        wait_loads(par)

    plsc.subcore_barrier()
    ro = c * NN + r0

    @pl.when(s < 15)
    def _():
        pltpu.sync_copy(agg_s.at[pl.ds(r0, RPT)], agg_out.at[pl.ds(ro, RPT)])
        pltpu.sync_copy(t_s.at[pl.ds(r0, RPT)], t_out.at[pl.ds(ro, RPT)])

    @pl.when(s == 15)
    def _():
        pltpu.sync_copy(agg_s.at[pl.ds(r0, LASTR)], agg_out.at[pl.ds(ro, LASTR)])
        pltpu.sync_copy(t_s.at[pl.ds(r0, LASTR)], t_out.at[pl.ds(ro, LASTR)])


@functools.lru_cache(maxsize=1)
def _get_sc_call():
  scr = [pltpu.VMEM_SHARED((NN, 32), F32),
         pltpu.VMEM_SHARED((NN, 8), F32)]
  scr += [pltpu.VMEM((B,), I32) for _ in range(14)]
  scr += [pltpu.VMEM((B,), F32) for _ in range(4)]
  scr += [pltpu.VMEM((B, 32), F32) for _ in range(2)]
  scr += [pltpu.VMEM((B, 8), F32) for _ in range(2)]
  scr += [pltpu.VMEM((16,), F32) for _ in range(2)]
  scr += [pltpu.SemaphoreType.DMA for _ in range(6)]
  return pl.kernel(
    _sc_body,
    out_type=(jax.ShapeDtypeStruct((2 * NN, 32), F32),
              jax.ShapeDtypeStruct((2 * NN, 8), F32)),
    mesh=plsc.VectorSubcoreMesh(core_axis_name="c", subcore_axis_name="s",
                                num_cores=2, num_subcores=16),
    compiler_params=pltpu.CompilerParams(needs_layout_passes=False,
                                         use_tc_tiling_on_sc=False),
    scratch_types=scr,
  )


def _layer_consts(eemb, aedge):
    er = eemb.reshape(5, 2, 32)
    se = jnp.sum(er * aedge[None, :, :], axis=2)
    secat = jnp.zeros((2, 16), F32).at[:, :5].set(se.T).reshape(32)
    eembt = jnp.zeros((2, 8, 32), F32).at[:, :5, :].set(er.transpose(1, 0, 2))
    return se, secat, eembt


def _mcat(bm, se):
    bmv = jnp.max(bm, axis=(0, 1))
    bound = bmv[:2] + bmv[2:] + jnp.max(se, axis=0)
    m = jnp.where(bound > 0, bound, 0.2 * bound)
    return jnp.broadcast_to(m[:, None], (2, 16)).reshape(32)


def kernel(x, edge_index, edge_attr,
           W0, eemb0, asrc0, adst0, aedge0,
           W1, eemb1, asrc1, adst1, aedge1,
           W2, eemb2, asrc2, adst2, aedge2,
           W3, eemb3, asrc3, adst3, aedge3):
    pad = EALLOC - EE
    zpad = jnp.zeros((pad,), I32)
    srcp = jnp.concatenate([edge_index[0], zpad])
    dstp = jnp.concatenate([edge_index[1], zpad])
    attrp = jnp.concatenate([edge_attr, zpad])
    za = jnp.zeros((RPT, 32), F32)
    zb = jnp.zeros((RPT, 8), F32)

    params = [(W0, eemb0, asrc0, adst0, aedge0),
              (W1, eemb1, asrc1, adst1, aedge1),
              (W2, eemb2, asrc2, adst2, aedge2),
              (W3, eemb3, asrc3, adst3, aedge3)]
    consts = [_layer_consts(e, ae) for (_, e, _, _, ae) in params]

    xp = jnp.pad(x, ((0, 0), (0, 4)))
    W0p = jnp.pad(W0, ((0, 4), (0, 0)))
    ht, ssc, sdc, bm = _dense0_call(xp, W0p, asrc0, adst0)

    sc_call = _get_sc_call()
    agg = t = None
    for i in range(4):
        se, secat, _ = consts[i]
        agg, t = sc_call(srcp, dstp, attrp,
                          ht.reshape(2 * NN, 32),
                          ssc.T.reshape(2 * NN),
                          sdc.T.reshape(2 * NN),
                          secat, _mcat(bm, se), za, zb)
        if i < 3:
            W = params[i + 1][0]
            asrc = params[i + 1][2]
            adst = params[i + 1][3]
            ht, ssc, sdc, bm = _denseN_call(agg, agg, t, t,
                                            consts[i][2], W, asrc, adst)

    pred = _final_call(agg, agg, agg, agg, t, t, t, t, consts[3][2])[0]
    return pred.reshape(HALF)
